# baseline (speedup 1.0000x reference)
import jax
import jax.numpy as jnp
from jax import lax
from jax.experimental import pallas as pl
from jax.experimental.pallas import tpu as pltpu
from jax.experimental.pallas import tpu_sc as plsc

MEMORY_SIZE = 1000000
NGRAM_N = 4
NUM_HEADS = 8
HEAD_DIM = 8
B, W, O = 4096, 50, 8
SEQ_W = 72
NC, NS = 2, 16
NW = NC * NS
BPW = B // NW
ROWS = W * NUM_HEADS


def _primes():
    ps = []
    base = 131
    for h in range(NUM_HEADS):
        x = base + h * 1009
        row = []
        for _ in range(NGRAM_N):
            row.append(x & 0xFFFFFFFF)
            x = x * 31 + 1
        ps.append(row)
    return ps


_P = _primes()


def _body(seq_hbm, tab_hbm, gate_hbm, out_hbm,
          seq_v, idx_v, stage_v, gate_v, sem):
    wid = lax.axis_index("s") * NC + lax.axis_index("c")
    b0 = wid * BPW

    pltpu.sync_copy(seq_hbm.at[pl.ds(b0, BPW), :], seq_v)
    pltpu.sync_copy(gate_hbm, gate_v)

    lanes = lax.iota(jnp.int32, 16)
    step = (lanes >= 8).astype(jnp.int32)
    colpat = lanes - 8 * step
    lanes8 = lanes * 8

    sig = []
    for q in range(4):
        g = gate_v[pl.ds(16 * q, 16)]
        sig.append(1.0 / (1.0 + jnp.exp(-g)))

    def per_row(b, carry):
        for c in range(4):
            wins = []
            for i in range(NGRAM_N):
                w = seq_v[b, pl.ds(O + 16 * c - i, 16)]
                wins.append(plsc.bitcast(w, jnp.uint32))
            for h in range(NUM_HEADS):
                acc = wins[0] * jnp.uint32(_P[h][0])
                for i in range(1, NGRAM_N):
                    acc = acc + wins[i] * jnp.uint32(_P[h][i])
                m = acc % jnp.uint32(MEMORY_SIZE)
                flat = plsc.bitcast(m * jnp.uint32(8) + jnp.uint32(h),
                                    jnp.int32)
                plsc.store_scatter(
                    idx_v,
                    [jnp.full((16,), c, jnp.int32), lanes8 + h],
                    flat)

        cps = []
        for c in range(3):
            cps.append(pltpu.async_copy(
                tab_hbm.at[idx_v.at[c]],
                stage_v.at[pl.ds(128 * c, 128), :], sem))
        cps.append(pltpu.async_copy(
            tab_hbm.at[idx_v.at[3, pl.ds(0, 16)]],
            stage_v.at[pl.ds(384, 16), :], sem))
        for cp in cps:
            cp.wait()

        def mul_j(j, inner):
            for q in range(4):
                row = step + (8 * j + 2 * q)
                v = plsc.load_gather(stage_v, [row, colpat])
                plsc.store_scatter(stage_v, [row, colpat], v * sig[q])
            return inner
        lax.fori_loop(0, W, mul_j, 0, unroll=False)

        pltpu.sync_copy(stage_v,
                        out_hbm.at[pl.ds((b0 + b) * ROWS, ROWS), :])
        return carry

    lax.fori_loop(0, BPW, per_row, 0, unroll=False)


@jax.jit
def kernel(current_ids, prev_ids_overlap, engram_table, gate_logit):
    seq = jnp.concatenate(
        [prev_ids_overlap, current_ids,
         jnp.zeros((B, SEQ_W - O - W), jnp.int32)], axis=1)
    tab = engram_table.reshape(MEMORY_SIZE * NUM_HEADS, HEAD_DIM)
    gate = gate_logit.reshape(NUM_HEADS * HEAD_DIM)
    mesh = plsc.VectorSubcoreMesh(core_axis_name="c", subcore_axis_name="s",
                                  num_cores=NC, num_subcores=NS)
    out = pl.kernel(
        _body,
        out_type=jax.ShapeDtypeStruct((B * ROWS, HEAD_DIM), jnp.float32),
        mesh=mesh,
        compiler_params=pltpu.CompilerParams(use_tc_tiling_on_sc=False,
                                             needs_layout_passes=False),
        scratch_types=[
            pltpu.VMEM((BPW, SEQ_W), jnp.int32),
            pltpu.VMEM((4, 128), jnp.int32),
            pltpu.VMEM((ROWS, HEAD_DIM), jnp.float32),
            pltpu.VMEM((NUM_HEADS * HEAD_DIM,), jnp.float32),
            pltpu.SemaphoreType.DMA,
        ],
    )(seq, tab, gate)
    return out.reshape(B, W, NUM_HEADS * HEAD_DIM)


# speedup vs baseline: 1.0439x; 1.0439x over previous
import jax
import jax.numpy as jnp
from jax import lax
from jax.experimental import pallas as pl
from jax.experimental.pallas import tpu as pltpu
from jax.experimental.pallas import tpu_sc as plsc

MEMORY_SIZE = 1000000
NGRAM_N = 4
NUM_HEADS = 8
HEAD_DIM = 8
B, W, O = 4096, 50, 8
SEQ_W = 72
NC, NS = 2, 16
NW = NC * NS
BPW = B // NW
ROWS = W * NUM_HEADS
G = 8
RND = BPW // G
GR = G * ROWS
IDXR = GR // 128


def _primes():
    ps = []
    base = 131
    for h in range(NUM_HEADS):
        x = base + h * 1009
        row = []
        for _ in range(NGRAM_N):
            row.append(x & 0xFFFFFFFF)
            x = x * 31 + 1
        ps.append(row)
    return ps


_P = _primes()


def _body(seq_hbm, tab_hbm, gate_hbm, out_hbm,
          seq_v, idx0, idx1, st0, st1, gate_v, sg0, sg1):
    wid = lax.axis_index("s") * NC + lax.axis_index("c")
    b0 = wid * BPW

    pltpu.sync_copy(seq_hbm.at[pl.ds(b0, BPW), :], seq_v)
    pltpu.sync_copy(gate_hbm, gate_v)

    lanes = lax.iota(jnp.int32, 16)
    step = (lanes >= 8).astype(jnp.int32)
    colpat = lanes - 8 * step
    lanes8 = lanes * 8
    tailmask = lanes < 2

    sig = []
    for q in range(4):
        g = gate_v[pl.ds(16 * q, 16)]
        sig.append(1.0 / (1.0 + jnp.exp(-g)))

    def hash_round(r, idx_s):
        def hb(bl, carry):
            b = r * G + bl
            ebase = bl * ROWS
            for c in range(4):
                wins = []
                for i in range(NGRAM_N):
                    w = seq_v[b, pl.ds(O + 16 * c - i, 16)]
                    wins.append(plsc.bitcast(w, jnp.uint32))
                for h in range(NUM_HEADS):
                    acc = wins[0] * jnp.uint32(_P[h][0])
                    for i in range(1, NGRAM_N):
                        acc = acc + wins[i] * jnp.uint32(_P[h][i])
                    m = acc % jnp.uint32(MEMORY_SIZE)
                    flat = plsc.bitcast(m * jnp.uint32(8) + jnp.uint32(h),
                                        jnp.int32)
                    e = lanes8 + (ebase + c * 128 + h)
                    row = lax.shift_right_logical(e, 7)
                    col = jnp.bitwise_and(e, 127)
                    if c < 3:
                        plsc.store_scatter(idx_s, [row, col], flat)
                    else:
                        plsc.store_scatter(idx_s, [row, col], flat,
                                           mask=tailmask)
            return carry
        lax.fori_loop(0, G, hb, 0, unroll=False)

    def gather_cps(idx_s, st_s, sem):
        return [pltpu.make_async_copy(
            tab_hbm.at[idx_s.at[k]],
            st_s.at[pl.ds(128 * k, 128), :], sem) for k in range(IDXR)]

    def mul_round(st_s):
        def mj(j, carry):
            for q in range(4):
                row = step + (8 * j + 2 * q)
                v = plsc.load_gather(st_s, [row, colpat])
                plsc.store_scatter(st_s, [row, colpat], v * sig[q])
            return carry
        lax.fori_loop(0, GR // 8, mj, 0, unroll=False)

    def out_copy(st_s, r):
        pltpu.sync_copy(st_s, out_hbm.at[pl.ds((b0 + G * r) * ROWS, GR), :])

    slots = [(idx0, st0, sg0), (idx1, st1, sg1)]

    def steady(r, s):
        idx_s, st_s, sg_s = slots[s]
        idx_n, st_n, sg_n = slots[1 - s]
        hash_round(r + 1, idx_n)
        for cp in gather_cps(idx_n, st_n, sg_n):
            cp.start()
        for cp in gather_cps(idx_s, st_s, sg_s):
            cp.wait()
        mul_round(st_s)
        out_copy(st_s, r)

    hash_round(0, idx0)
    for cp in gather_cps(idx0, st0, sg0):
        cp.start()

    def pair(p, carry):
        steady(2 * p, 0)
        steady(2 * p + 1, 1)
        return carry
    lax.fori_loop(0, (RND - 2) // 2, pair, 0, unroll=False)

    steady(RND - 2, 0)
    idx_s, st_s, sg_s = slots[1]
    for cp in gather_cps(idx_s, st_s, sg_s):
        cp.wait()
    mul_round(st_s)
    out_copy(st_s, RND - 1)


@jax.jit
def kernel(current_ids, prev_ids_overlap, engram_table, gate_logit):
    seq = jnp.concatenate(
        [prev_ids_overlap, current_ids,
         jnp.zeros((B, SEQ_W - O - W), jnp.int32)], axis=1)
    tab = engram_table.reshape(MEMORY_SIZE * NUM_HEADS, HEAD_DIM)
    gate = gate_logit.reshape(NUM_HEADS * HEAD_DIM)
    mesh = plsc.VectorSubcoreMesh(core_axis_name="c", subcore_axis_name="s",
                                  num_cores=NC, num_subcores=NS)
    out = pl.kernel(
        _body,
        out_type=jax.ShapeDtypeStruct((B * ROWS, HEAD_DIM), jnp.float32),
        mesh=mesh,
        compiler_params=pltpu.CompilerParams(use_tc_tiling_on_sc=False,
                                             needs_layout_passes=False),
        scratch_types=[
            pltpu.VMEM((BPW, SEQ_W), jnp.int32),
            pltpu.VMEM((IDXR, 128), jnp.int32),
            pltpu.VMEM((IDXR, 128), jnp.int32),
            pltpu.VMEM((GR, HEAD_DIM), jnp.float32),
            pltpu.VMEM((GR, HEAD_DIM), jnp.float32),
            pltpu.VMEM((NUM_HEADS * HEAD_DIM,), jnp.float32),
            pltpu.SemaphoreType.DMA,
            pltpu.SemaphoreType.DMA,
        ],
    )(seq, tab, gate)
    return out.reshape(B, W, NUM_HEADS * HEAD_DIM)
